# baseline (speedup 1.0000x reference)
import functools

import jax
import jax.numpy as jnp
from jax.experimental import pallas as pl
from jax.experimental.pallas import tpu as pltpu

ALPHA = 5.0
EPS = 1e-8


def _partial_loss_kernel(labels_ref, x_ref, out_ref, *, n_cols, block_rows):
    x = x_ref[...]
    ss = jnp.sum(x * x, axis=1, keepdims=True)
    lcol = jax.lax.broadcasted_iota(jnp.int32, (block_rows, n_cols), 1)
    g = jnp.sum(jnp.where(lcol == labels_ref[...], x, 0.0), axis=1,
                keepdims=True)
    s = g / jnp.maximum(jnp.sqrt(ss), EPS)
    loss_terms = (1.0 - s) / (1.0 + s) * ALPHA
    out_ref[0, 0, 0] = jnp.sum(loss_terms)


def _reduce_kernel(p_ref, out_ref, *, n_rows):
    out_ref[0, 0] = jnp.sum(p_ref[...]) / n_rows


def kernel(logits, labels):
    n_rows, n_cols = logits.shape
    block_rows = 32
    n_blocks = n_rows // block_rows
    labels2 = labels.astype(jnp.int32).reshape(n_rows, 1)

    partials = pl.pallas_call(
        functools.partial(
            _partial_loss_kernel, n_cols=n_cols, block_rows=block_rows),
        grid=(n_blocks,),
        in_specs=[
            pl.BlockSpec((block_rows, 1), lambda rb: (rb, 0)),
            pl.BlockSpec((block_rows, n_cols), lambda rb: (rb, 0)),
        ],
        out_specs=pl.BlockSpec(
            (1, 1, 1), lambda rb: (rb, 0, 0), memory_space=pltpu.SMEM),
        out_shape=jax.ShapeDtypeStruct((n_blocks, 1, 1), jnp.float32),
        compiler_params=pltpu.CompilerParams(
            dimension_semantics=("parallel",)),
    )(labels2, logits)

    out = pl.pallas_call(
        functools.partial(_reduce_kernel, n_rows=n_rows),
        out_specs=pl.BlockSpec(memory_space=pltpu.SMEM),
        out_shape=jax.ShapeDtypeStruct((1, 1), jnp.float32),
    )(partials)
    return out[0, 0]
